# baseline (speedup 1.0000x reference)
import functools

import jax
import jax.numpy as jnp
from jax import lax
from jax.experimental import pallas as pl
from jax.experimental.pallas import tpu as pltpu
from jax.experimental.pallas import tpu_sc as plsc

N = 10000
E = 320000
D_IN = 128
D_H = 256
N_CLS = 64

NC = 2
NS = 16
NW = NC * NS

CW = 16
C = 125
EPW = E // NW
K = EPW // C

ROWS_A = 640
ROWS_B = N - ROWS_A * (NS - 1)
ZR = 16


def _make_agg(D):
    mesh = plsc.VectorSubcoreMesh(core_axis_name="c", subcore_axis_name="s",
                                  num_cores=NC, num_subcores=NS)

    @functools.partial(
        pl.kernel,
        out_type=(jax.ShapeDtypeStruct((NC, N, D), jnp.float32),
                  jax.ShapeDtypeStruct((NC, N, CW), jnp.float32)),
        mesh=mesh,
        scratch_types=[
            pltpu.VMEM_SHARED((N, D), jnp.float32),
            pltpu.VMEM_SHARED((N, CW), jnp.float32),
            pltpu.VMEM((C,), jnp.int32),
            pltpu.VMEM((C,), jnp.int32),
            pltpu.VMEM((C,), jnp.int32),
            pltpu.VMEM((C,), jnp.int32),
            pltpu.VMEM((C, D), jnp.float32),
            pltpu.VMEM((C, D), jnp.float32),
            pltpu.VMEM((C, CW), jnp.float32),
            pltpu.VMEM((ZR, D), jnp.float32),
            pltpu.VMEM((ZR, CW), jnp.float32),
            pltpu.SemaphoreType.DMA,
            pltpu.SemaphoreType.DMA,
            pltpu.SemaphoreType.DMA,
            pltpu.SemaphoreType.DMA,
            pltpu.SemaphoreType.DMA,
            pltpu.SemaphoreType.DMA,
            pltpu.SemaphoreType.DMA,
            pltpu.SemaphoreType.DMA,
            pltpu.SemaphoreType.DMA,
        ],
        compiler_params=pltpu.CompilerParams(use_tc_tiling_on_sc=False),
    )
    def agg(feat, src2, dst2, sums_out, cnt_out, acc, cnt,
            sidx_a, sidx_b, didx_a, didx_b, rows_a, rows_b,
            ones_blk, zrow, zcnt,
            sem_a, sem_b, sem_sa, sem_sb,
            sem_si_a, sem_si_b, sem_di_a, sem_di_b, sem_z):
        cid = lax.axis_index("c")
        sid = lax.axis_index("s")
        wid = cid * NS + sid

        z16 = jnp.zeros((16,), jnp.float32)
        o16 = jnp.ones((16,), jnp.float32)

        def zfill(i, _):
            r = i // (D // 16)
            col = (i % (D // 16)) * 16
            zrow[r, pl.ds(col, 16)] = z16
            return 0

        lax.fori_loop(0, ZR * (D // 16), zfill, 0)

        def zcfill(i, _):
            zcnt[i, pl.ds(0, CW)] = z16[:CW]
            return 0

        lax.fori_loop(0, ZR, zcfill, 0)

        def ofill(i, _):
            ones_blk[i, pl.ds(0, CW)] = o16[:CW]
            return 0

        lax.fori_loop(0, C, ofill, 0)

        def zero_range(tile_base, nk):
            def grp(g, _):
                kb = g * 8

                def zs(k, _):
                    base = tile_base + (kb + k) * ZR
                    pltpu.async_copy(zrow, acc.at[pl.ds(base, ZR)], sem_z)
                    pltpu.async_copy(zcnt, cnt.at[pl.ds(base, ZR)], sem_z)
                    return 0

                def zw(k, _):
                    base = tile_base + (kb + k) * ZR
                    pltpu.make_async_copy(zrow, acc.at[pl.ds(base, ZR)],
                                          sem_z).wait()
                    pltpu.make_async_copy(zcnt, cnt.at[pl.ds(base, ZR)],
                                          sem_z).wait()
                    return 0

                lax.fori_loop(0, 8, zs, 0)
                lax.fori_loop(0, 8, zw, 0)
                return 0
            lax.fori_loop(0, nk // 8, grp, 0)

        @pl.when(sid < NS - 1)
        def _():
            zero_range(sid * ROWS_A, ROWS_A // ZR)

        @pl.when(sid == NS - 1)
        def _():
            zero_range((NS - 1) * ROWS_A, 24)
            base = (NS - 1) * ROWS_A + 24 * ZR
            pltpu.sync_copy(zrow, acc.at[pl.ds(base, ZR)])
            pltpu.sync_copy(zcnt, cnt.at[pl.ds(base, ZR)])

        plsc.subcore_barrier()

        def g_start(sidx, rows, sem):
            pltpu.async_copy(feat.at[sidx], rows, sem)

        def g_wait(sidx, rows, sem):
            pltpu.make_async_copy(feat.at[sidx], rows, sem).wait()

        def s_start(rows, didx, sem):
            pltpu.async_copy(rows, acc.at[didx], sem, add=True)
            pltpu.async_copy(ones_blk, cnt.at[didx], sem, add=True)

        def s_wait(rows, didx, sem):
            pltpu.make_async_copy(rows, acc.at[didx], sem).wait()
            pltpu.make_async_copy(ones_blk, cnt.at[didx], sem).wait()

        pltpu.sync_copy(src2.at[wid, 0], sidx_a)
        g_start(sidx_a, rows_a, sem_a)
        pltpu.async_copy(src2.at[wid, 1], sidx_b, sem_si_b)
        pltpu.async_copy(dst2.at[wid, 0], didx_a, sem_di_a)
        pltpu.async_copy(dst2.at[wid, 1], didx_b, sem_di_b)

        def ebody(i, _):
            j = 2 * i
            g_wait(sidx_a, rows_a, sem_a)

            @pl.when(j + 2 < K)
            def _():
                pltpu.async_copy(src2.at[wid, j + 2], sidx_a, sem_si_a)

            pltpu.make_async_copy(dst2.at[wid, j], didx_a, sem_di_a).wait()
            s_start(rows_a, didx_a, sem_sa)

            pltpu.make_async_copy(src2.at[wid, j + 1], sidx_b, sem_si_b).wait()

            @pl.when(i > 0)
            def _():
                s_wait(rows_b, didx_b, sem_sb)
                pltpu.async_copy(dst2.at[wid, j + 1], didx_b, sem_di_b)

            g_start(sidx_b, rows_b, sem_b)

            s_wait(rows_a, didx_a, sem_sa)

            @pl.when(j + 2 < K)
            def _():
                pltpu.async_copy(dst2.at[wid, j + 2], didx_a, sem_di_a)
                pltpu.make_async_copy(src2.at[wid, j + 2], sidx_a,
                                      sem_si_a).wait()
                g_start(sidx_a, rows_a, sem_a)

            g_wait(sidx_b, rows_b, sem_b)

            @pl.when(j + 3 < K)
            def _():
                pltpu.async_copy(src2.at[wid, j + 3], sidx_b, sem_si_b)

            pltpu.make_async_copy(dst2.at[wid, j + 1], didx_b, sem_di_b).wait()
            s_start(rows_b, didx_b, sem_sb)
            return 0

        lax.fori_loop(0, K // 2, ebody, 0)
        s_wait(rows_b, didx_b, sem_sb)

        plsc.subcore_barrier()

        def wout(base, nrows_tag):
            if nrows_tag == 0:
                nr = ROWS_A
            else:
                nr = ROWS_B
            pltpu.async_copy(acc.at[pl.ds(base, nr)],
                             sums_out.at[cid, pl.ds(base, nr)], sem_z)
            pltpu.async_copy(cnt.at[pl.ds(base, nr)],
                             cnt_out.at[cid, pl.ds(base, nr)], sem_z)
            pltpu.make_async_copy(acc.at[pl.ds(base, nr)],
                                  sums_out.at[cid, pl.ds(base, nr)],
                                  sem_z).wait()
            pltpu.make_async_copy(cnt.at[pl.ds(base, nr)],
                                  cnt_out.at[cid, pl.ds(base, nr)],
                                  sem_z).wait()

        @pl.when(sid < NS - 1)
        def _():
            wout(sid * ROWS_A, 0)

        @pl.when(sid == NS - 1)
        def _():
            wout((NS - 1) * ROWS_A, 1)

    return agg


_agg_l0 = _make_agg(D_IN)
_agg_l1 = _make_agg(N_CLS)

_R1 = 1000
_R2 = 1000


def _tc_fuse(p0, c0, W0, b0, W1):
    def body(p_ref, c_ref, w0_ref, b0_ref, w1_ref, z_ref):
        a = p_ref[0] + p_ref[1]
        c16 = c_ref[0] + c_ref[1]
        c = c16[:, 0:1]
        inv = 1.0 / jnp.maximum(c, 1.0)
        mask = (c > 0.0).astype(jnp.float32)
        feats = a * inv
        h = jnp.dot(feats, w0_ref[...], preferred_element_type=jnp.float32)
        h = jnp.maximum(h + b0_ref[...] * mask, 0.0)
        z_ref[...] = jnp.dot(h, w1_ref[...], preferred_element_type=jnp.float32)

    return pl.pallas_call(
        body,
        grid=(N // _R1,),
        in_specs=[
            pl.BlockSpec((NC, _R1, D_IN), lambda i: (0, i, 0)),
            pl.BlockSpec((NC, _R1, CW), lambda i: (0, i, 0)),
            pl.BlockSpec((D_IN, D_H), lambda i: (0, 0)),
            pl.BlockSpec((1, D_H), lambda i: (0, 0)),
            pl.BlockSpec((D_H, N_CLS), lambda i: (0, 0)),
        ],
        out_specs=pl.BlockSpec((_R1, N_CLS), lambda i: (i, 0)),
        out_shape=jax.ShapeDtypeStruct((N, N_CLS), jnp.float32),
    )(p0, c0, W0, b0, W1)


def _tc_final(p1, c1, b1):
    def body(p_ref, c_ref, b_ref, o_ref):
        s = p_ref[0] + p_ref[1]
        c16 = c_ref[0] + c_ref[1]
        c = c16[:, 0:1]
        inv = 1.0 / jnp.maximum(c, 1.0)
        mask = (c > 0.0).astype(jnp.float32)
        o_ref[...] = s * inv + b_ref[...] * mask

    return pl.pallas_call(
        body,
        grid=(N // _R2,),
        in_specs=[
            pl.BlockSpec((NC, _R2, N_CLS), lambda i: (0, i, 0)),
            pl.BlockSpec((NC, _R2, CW), lambda i: (0, i, 0)),
            pl.BlockSpec((1, N_CLS), lambda i: (0, 0)),
        ],
        out_specs=pl.BlockSpec((_R2, N_CLS), lambda i: (i, 0)),
        out_shape=jax.ShapeDtypeStruct((N, N_CLS), jnp.float32),
    )(p1, c1, b1)


def kernel(x, edge_index0, edge_index1, W0, b0, W1, b1):
    src0 = edge_index0[0].reshape(NW, K, C)
    dst0 = edge_index0[1].reshape(NW, K, C)
    src1 = edge_index1[0].reshape(NW, K, C)
    dst1 = edge_index1[1].reshape(NW, K, C)

    p0, c0 = _agg_l0(x, src0, dst0)
    z = _tc_fuse(p0, c0, W0, b0.reshape(1, D_H), W1)
    p1, c1 = _agg_l1(z, src1, dst1)
    return _tc_final(p1, c1, b1.reshape(1, N_CLS))


# speedup vs baseline: 1.0145x; 1.0145x over previous
import functools

import jax
import jax.numpy as jnp
from jax import lax
from jax.experimental import pallas as pl
from jax.experimental.pallas import tpu as pltpu
from jax.experimental.pallas import tpu_sc as plsc

N = 10000
E = 320000
D_IN = 128
D_H = 256
N_CLS = 64

NC = 2
NS = 16
NW = NC * NS

CW = 16
CH = 128
NCHUNK = E // CH
KPW = NCHUNK // NW
NEXTRA = NCHUNK - KPW * NW
PACK = 16384

ROWS_A = 640
ROWS_B = N - ROWS_A * (NS - 1)
ZR = 16


def _make_agg(D):
    mesh = plsc.VectorSubcoreMesh(core_axis_name="c", subcore_axis_name="s",
                                  num_cores=NC, num_subcores=NS)

    @functools.partial(
        pl.kernel,
        out_type=(jax.ShapeDtypeStruct((NC, N, D), jnp.float32),
                  jax.ShapeDtypeStruct((NC, N, CW), jnp.float32)),
        mesh=mesh,
        scratch_types=[
            pltpu.VMEM_SHARED((N, D), jnp.float32),
            pltpu.VMEM_SHARED((N, CW), jnp.float32),
            pltpu.VMEM((CH,), jnp.int32),
            pltpu.VMEM((CH,), jnp.int32),
            pltpu.VMEM((CH,), jnp.int32),
            pltpu.VMEM((CH,), jnp.int32),
            pltpu.VMEM((CH,), jnp.int32),
            pltpu.VMEM((CH,), jnp.int32),
            pltpu.VMEM((CH, D), jnp.float32),
            pltpu.VMEM((CH, D), jnp.float32),
            pltpu.VMEM((CH, CW), jnp.float32),
            pltpu.VMEM((ZR, D), jnp.float32),
            pltpu.VMEM((ZR, CW), jnp.float32),
            pltpu.SemaphoreType.DMA,
            pltpu.SemaphoreType.DMA,
            pltpu.SemaphoreType.DMA,
            pltpu.SemaphoreType.DMA,
            pltpu.SemaphoreType.DMA,
            pltpu.SemaphoreType.DMA,
            pltpu.SemaphoreType.DMA,
        ],
        compiler_params=pltpu.CompilerParams(use_tc_tiling_on_sc=False),
    )
    def agg(feat, pidx, sums_out, cnt_out, acc, cnt,
            raw_a, raw_b, sidx_a, sidx_b, didx_a, didx_b, rows_a, rows_b,
            ones_blk, zrow, zcnt,
            sem_a, sem_b, sem_sa, sem_sb, sem_ia, sem_ib, sem_z):
        cid = lax.axis_index("c")
        sid = lax.axis_index("s")
        wid = cid * NS + sid
        wbase = wid * KPW

        z16 = jnp.zeros((16,), jnp.float32)
        o16 = jnp.ones((16,), jnp.float32)

        def zfill(i, _):
            r = i // (D // 16)
            col = (i % (D // 16)) * 16
            zrow[r, pl.ds(col, 16)] = z16
            return 0

        lax.fori_loop(0, ZR * (D // 16), zfill, 0)

        def zcfill(i, _):
            zcnt[i, pl.ds(0, CW)] = z16[:CW]
            return 0

        lax.fori_loop(0, ZR, zcfill, 0)

        def ofill(i, _):
            ones_blk[i, pl.ds(0, CW)] = o16[:CW]
            return 0

        lax.fori_loop(0, CH, ofill, 0)

        def zero_range(tile_base, nk):
            def grp(g, _):
                kb = g * 8

                def zs(k, _):
                    base = tile_base + (kb + k) * ZR
                    pltpu.async_copy(zrow, acc.at[pl.ds(base, ZR)], sem_z)
                    pltpu.async_copy(zcnt, cnt.at[pl.ds(base, ZR)], sem_z)
                    return 0

                def zw(k, _):
                    base = tile_base + (kb + k) * ZR
                    pltpu.make_async_copy(zrow, acc.at[pl.ds(base, ZR)],
                                          sem_z).wait()
                    pltpu.make_async_copy(zcnt, cnt.at[pl.ds(base, ZR)],
                                          sem_z).wait()
                    return 0

                lax.fori_loop(0, 8, zs, 0)
                lax.fori_loop(0, 8, zw, 0)
                return 0
            lax.fori_loop(0, nk // 8, grp, 0)

        @pl.when(sid < NS - 1)
        def _():
            zero_range(sid * ROWS_A, ROWS_A // ZR)

        @pl.when(sid == NS - 1)
        def _():
            zero_range((NS - 1) * ROWS_A, 24)
            base = (NS - 1) * ROWS_A + 24 * ZR
            pltpu.sync_copy(zrow, acc.at[pl.ds(base, ZR)])
            pltpu.sync_copy(zcnt, cnt.at[pl.ds(base, ZR)])

        plsc.subcore_barrier()

        def unpack(raw, sidx, didx):
            def ub(k, _):
                v = raw[pl.ds(k * 16, 16)]
                sidx[pl.ds(k * 16, 16)] = lax.shift_right_logical(v, 14)
                didx[pl.ds(k * 16, 16)] = lax.bitwise_and(v, PACK - 1)
                return 0
            lax.fori_loop(0, CH // 16, ub, 0)

        def g_start(sidx, rows, sem):
            pltpu.async_copy(feat.at[sidx], rows, sem)

        def g_wait(sidx, rows, sem):
            pltpu.make_async_copy(feat.at[sidx], rows, sem).wait()

        def s_start(rows, didx, sem):
            pltpu.async_copy(rows, acc.at[didx], sem, add=True)
            pltpu.async_copy(ones_blk, cnt.at[didx], sem, add=True)

        def s_wait(rows, didx, sem):
            pltpu.make_async_copy(rows, acc.at[didx], sem).wait()
            pltpu.make_async_copy(ones_blk, cnt.at[didx], sem).wait()

        pltpu.sync_copy(pidx.at[wbase], raw_a)
        unpack(raw_a, sidx_a, didx_a)
        g_start(sidx_a, rows_a, sem_a)
        pltpu.async_copy(pidx.at[wbase + 1], raw_b, sem_ib)

        def ebody(i, _):
            j = 2 * i
            g_wait(sidx_a, rows_a, sem_a)
            s_start(rows_a, didx_a, sem_sa)

            @pl.when(j + 2 < KPW)
            def _():
                pltpu.async_copy(pidx.at[wbase + j + 2], raw_a, sem_ia)

            pltpu.make_async_copy(pidx.at[wbase + j + 1], raw_b, sem_ib).wait()

            @pl.when(i > 0)
            def _():
                s_wait(rows_b, didx_b, sem_sb)

            unpack(raw_b, sidx_b, didx_b)
            g_start(sidx_b, rows_b, sem_b)

            s_wait(rows_a, didx_a, sem_sa)

            @pl.when(j + 2 < KPW)
            def _():
                pltpu.make_async_copy(pidx.at[wbase + j + 2], raw_a,
                                      sem_ia).wait()
                unpack(raw_a, sidx_a, didx_a)
                g_start(sidx_a, rows_a, sem_a)

            g_wait(sidx_b, rows_b, sem_b)
            s_start(rows_b, didx_b, sem_sb)

            @pl.when(j + 3 < KPW)
            def _():
                pltpu.async_copy(pidx.at[wbase + j + 3], raw_b, sem_ib)
            return 0

        lax.fori_loop(0, KPW // 2, ebody, 0)
        s_wait(rows_b, didx_b, sem_sb)

        @pl.when(wid < NEXTRA)
        def _():
            gx = NW * KPW + wid
            pltpu.sync_copy(pidx.at[gx], raw_a)
            unpack(raw_a, sidx_a, didx_a)
            g_start(sidx_a, rows_a, sem_a)
            g_wait(sidx_a, rows_a, sem_a)
            s_start(rows_a, didx_a, sem_sa)
            s_wait(rows_a, didx_a, sem_sa)

        plsc.subcore_barrier()

        def wout(base, nrows_tag):
            if nrows_tag == 0:
                nr = ROWS_A
            else:
                nr = ROWS_B
            pltpu.async_copy(acc.at[pl.ds(base, nr)],
                             sums_out.at[cid, pl.ds(base, nr)], sem_z)
            pltpu.async_copy(cnt.at[pl.ds(base, nr)],
                             cnt_out.at[cid, pl.ds(base, nr)], sem_z)
            pltpu.make_async_copy(acc.at[pl.ds(base, nr)],
                                  sums_out.at[cid, pl.ds(base, nr)],
                                  sem_z).wait()
            pltpu.make_async_copy(cnt.at[pl.ds(base, nr)],
                                  cnt_out.at[cid, pl.ds(base, nr)],
                                  sem_z).wait()

        @pl.when(sid < NS - 1)
        def _():
            wout(sid * ROWS_A, 0)

        @pl.when(sid == NS - 1)
        def _():
            wout((NS - 1) * ROWS_A, 1)

    return agg


_agg_l0 = _make_agg(D_IN)
_agg_l1 = _make_agg(N_CLS)

_R1 = 1000
_R2 = 1000


def _tc_fuse(p0, c0, W0, b0, W1):
    def body(p_ref, c_ref, w0_ref, b0_ref, w1_ref, z_ref):
        a = p_ref[0] + p_ref[1]
        c16 = c_ref[0] + c_ref[1]
        c = c16[:, 0:1]
        inv = 1.0 / jnp.maximum(c, 1.0)
        mask = (c > 0.0).astype(jnp.float32)
        feats = a * inv
        h = jnp.dot(feats, w0_ref[...], preferred_element_type=jnp.float32)
        h = jnp.maximum(h + b0_ref[...] * mask, 0.0)
        z_ref[...] = jnp.dot(h, w1_ref[...], preferred_element_type=jnp.float32)

    return pl.pallas_call(
        body,
        grid=(N // _R1,),
        in_specs=[
            pl.BlockSpec((NC, _R1, D_IN), lambda i: (0, i, 0)),
            pl.BlockSpec((NC, _R1, CW), lambda i: (0, i, 0)),
            pl.BlockSpec((D_IN, D_H), lambda i: (0, 0)),
            pl.BlockSpec((1, D_H), lambda i: (0, 0)),
            pl.BlockSpec((D_H, N_CLS), lambda i: (0, 0)),
        ],
        out_specs=pl.BlockSpec((_R1, N_CLS), lambda i: (i, 0)),
        out_shape=jax.ShapeDtypeStruct((N, N_CLS), jnp.float32),
    )(p0, c0, W0, b0, W1)


def _tc_final(p1, c1, b1):
    def body(p_ref, c_ref, b_ref, o_ref):
        s = p_ref[0] + p_ref[1]
        c16 = c_ref[0] + c_ref[1]
        c = c16[:, 0:1]
        inv = 1.0 / jnp.maximum(c, 1.0)
        mask = (c > 0.0).astype(jnp.float32)
        o_ref[...] = s * inv + b_ref[...] * mask

    return pl.pallas_call(
        body,
        grid=(N // _R2,),
        in_specs=[
            pl.BlockSpec((NC, _R2, N_CLS), lambda i: (0, i, 0)),
            pl.BlockSpec((NC, _R2, CW), lambda i: (0, i, 0)),
            pl.BlockSpec((1, N_CLS), lambda i: (0, 0)),
        ],
        out_specs=pl.BlockSpec((_R2, N_CLS), lambda i: (i, 0)),
        out_shape=jax.ShapeDtypeStruct((N, N_CLS), jnp.float32),
    )(p1, c1, b1)


def kernel(x, edge_index0, edge_index1, W0, b0, W1, b1):
    pidx0 = (edge_index0[0] * PACK + edge_index0[1]).reshape(NCHUNK, CH)
    pidx1 = (edge_index1[0] * PACK + edge_index1[1]).reshape(NCHUNK, CH)

    p0, c0 = _agg_l0(x, pidx0)
    z = _tc_fuse(p0, c0, W0, b0.reshape(1, D_H), W1)
    p1, c1 = _agg_l1(z, pidx1)
    return _tc_final(p1, c1, b1.reshape(1, N_CLS))
